# baseline (speedup 1.0000x reference)
import functools

import jax
import jax.numpy as jnp
from jax import lax
from jax.experimental import pallas as pl
from jax.experimental.pallas import tpu as pltpu
from jax.experimental.pallas import tpu_sc as plsc

_NUM_FIELDS = 26
_VOCAB = 100000
_BATCH = 16384
_LANES = 16
_NC, _NS = 2, 16

_mesh = plsc.VectorSubcoreMesh(
    core_axis_name="c", subcore_axis_name="s", num_cores=_NC, num_subcores=_NS
)


@functools.partial(
    pl.kernel,
    out_type=jax.ShapeDtypeStruct((_NUM_FIELDS * _BATCH,), jnp.float32),
    mesh=_mesh,
    scratch_types=[
        pltpu.VMEM((_VOCAB,), jnp.float32),
        pltpu.VMEM((_BATCH,), jnp.float32),
    ],
    compiler_params=pltpu.CompilerParams(needs_layout_passes=False),
)
def _gather_fields(tables_hbm, xbits_hbm, partial_hbm, table_v, buf_v):
    wid = lax.axis_index("s") * _NC + lax.axis_index("c")

    @pl.when(wid < _NUM_FIELDS)
    def _():
        pltpu.sync_copy(tables_hbm.at[pl.ds(wid * _VOCAB, _VOCAB)], table_v)
        pltpu.sync_copy(xbits_hbm.at[pl.ds(wid * _BATCH, _BATCH)], buf_v)

        def body(i, carry):
            sl = pl.ds(i * _LANES, _LANES)
            idx = buf_v[sl].astype(jnp.int32)
            buf_v[sl] = plsc.load_gather(table_v, [idx])
            return carry

        lax.fori_loop(0, _BATCH // _LANES, body, 0, unroll=8)
        pltpu.sync_copy(buf_v, partial_hbm.at[pl.ds(wid * _BATCH, _BATCH)])


def _combine_body(p_ref, w_ref, b_ref, o_ref):
    p = p_ref[...]
    w = w_ref[...]
    o_ref[...] = jax.nn.sigmoid(jnp.sum(p * w, axis=0, keepdims=True) + b_ref[...])


_combine = pl.pallas_call(
    _combine_body,
    out_shape=jax.ShapeDtypeStruct((1, _BATCH), jnp.float32),
)


def kernel(x, tables, weight, bias):
    xf = x.astype(jnp.float32).reshape(-1)
    tables_flat = tables.reshape(-1)
    partial = _gather_fields(tables_flat, xf)
    w = weight.reshape(_NUM_FIELDS, 1)
    b = bias.reshape(1, 1)
    return _combine(partial.reshape(_NUM_FIELDS, _BATCH), w, b)[0]


# speedup vs baseline: 3.0744x; 3.0744x over previous
import functools

import jax
import jax.numpy as jnp
from jax import lax
from jax.experimental import pallas as pl
from jax.experimental.pallas import tpu as pltpu
from jax.experimental.pallas import tpu_sc as plsc

_NUM_FIELDS = 26
_VOCAB = 100000
_BATCH = 16384
_LANES = 16
_HALF = _BATCH // 2
_NC, _NS = 2, 16

_mesh = plsc.VectorSubcoreMesh(
    core_axis_name="c", subcore_axis_name="s", num_cores=_NC, num_subcores=_NS
)


@functools.partial(
    pl.kernel,
    out_type=jax.ShapeDtypeStruct((_NUM_FIELDS, _BATCH), jnp.float32),
    mesh=_mesh,
    scratch_types=[
        pltpu.VMEM((_VOCAB,), jnp.float32),
        pltpu.VMEM((_BATCH,), jnp.int32),
        pltpu.VMEM((_HALF,), jnp.float32),
        pltpu.SemaphoreType.DMA,
        pltpu.SemaphoreType.DMA,
    ],
    compiler_params=pltpu.CompilerParams(needs_layout_passes=False),
)
def _gather_fields(tables_hbm, x_hbm, partial_hbm, table_v, idx_v, out_v, sem_t, sem_x):
    wid = lax.axis_index("s") * _NC + lax.axis_index("c")

    @pl.when(wid < _NUM_FIELDS)
    def _():
        cp_t = pltpu.async_copy(tables_hbm.at[wid], table_v, sem_t)
        cp_x = pltpu.async_copy(x_hbm.at[wid], idx_v, sem_x)
        cp_t.wait()
        cp_x.wait()

        def half(h):
            base = h * _HALF

            def body(i, carry):
                idx = idx_v[pl.ds(base + i * _LANES, _LANES)]
                out_v[pl.ds(i * _LANES, _LANES)] = plsc.load_gather(table_v, [idx])
                return carry

            lax.fori_loop(0, _HALF // _LANES, body, 0, unroll=8)
            pltpu.sync_copy(out_v, partial_hbm.at[wid, pl.ds(base, _HALF)])

        half(0)
        half(1)


def _combine_body(p_ref, w_ref, b_ref, o_ref):
    p = p_ref[...]
    w = w_ref[...]
    o_ref[...] = jax.nn.sigmoid(jnp.sum(p * w, axis=0, keepdims=True) + b_ref[...])


_combine = pl.pallas_call(
    _combine_body,
    out_shape=jax.ShapeDtypeStruct((1, _BATCH), jnp.float32),
)


def kernel(x, tables, weight, bias):
    tables2d = tables.reshape(_NUM_FIELDS, _VOCAB)
    partial = _gather_fields(tables2d, x.astype(jnp.int32))
    w = weight.reshape(_NUM_FIELDS, 1)
    b = bias.reshape(1, 1)
    return _combine(partial, w, b)[0]


# speedup vs baseline: 3.6049x; 1.1726x over previous
import functools

import jax
import jax.numpy as jnp
from jax import lax
from jax.experimental import pallas as pl
from jax.experimental.pallas import tpu as pltpu
from jax.experimental.pallas import tpu_sc as plsc

_NUM_FIELDS = 26
_VOCAB = 100000
_BATCH = 16384
_LANES = 16
_HALF = _BATCH // 2
_NC, _NS = 2, 16

_mesh = plsc.VectorSubcoreMesh(
    core_axis_name="c", subcore_axis_name="s", num_cores=_NC, num_subcores=_NS
)


@functools.partial(
    pl.kernel,
    out_type=jax.ShapeDtypeStruct((_NUM_FIELDS, _BATCH), jnp.float32),
    mesh=_mesh,
    scratch_types=[
        pltpu.VMEM((_VOCAB,), jnp.float32),
        pltpu.VMEM((_BATCH,), jnp.int32),
        pltpu.VMEM((_HALF,), jnp.float32),
        pltpu.SemaphoreType.DMA,
        pltpu.SemaphoreType.DMA,
    ],
    compiler_params=pltpu.CompilerParams(needs_layout_passes=False),
)
def _gather_fields(tables_hbm, x_hbm, partial_hbm, table_v, idx_v, out_v, sem_t, sem_x):
    wid = lax.axis_index("s") * _NC + lax.axis_index("c")

    @pl.when(wid < _NUM_FIELDS)
    def _():
        cp_t = pltpu.async_copy(tables_hbm.at[wid], table_v, sem_t)
        cp_x = pltpu.async_copy(x_hbm.at[wid], idx_v, sem_x)
        cp_t.wait()
        cp_x.wait()

        def half(h):
            base = h * _HALF

            @plsc.parallel_loop(0, _HALF // _LANES, 1, unroll=8)
            def _loop(i):
                idx = idx_v[pl.ds(base + i * _LANES, _LANES)]
                out_v[pl.ds(i * _LANES, _LANES)] = plsc.load_gather(table_v, [idx])

            pltpu.sync_copy(out_v, partial_hbm.at[wid, pl.ds(base, _HALF)])

        half(0)
        half(1)


def _combine_body(p_ref, w_ref, b_ref, o_ref):
    p = p_ref[...]
    w = w_ref[...]
    o_ref[...] = jax.nn.sigmoid(jnp.sum(p * w, axis=0, keepdims=True) + b_ref[...])


_combine = pl.pallas_call(
    _combine_body,
    out_shape=jax.ShapeDtypeStruct((1, _BATCH), jnp.float32),
)


def kernel(x, tables, weight, bias):
    tables2d = tables.reshape(_NUM_FIELDS, _VOCAB)
    partial = _gather_fields(tables2d, x.astype(jnp.int32))
    w = weight.reshape(_NUM_FIELDS, 1)
    b = bias.reshape(1, 1)
    return _combine(partial, w, b)[0]
